# baseline (speedup 1.0000x reference)
import jax
import jax.numpy as jnp
from jax import lax
from jax.experimental import pallas as pl
from jax.experimental.pallas import tpu as pltpu
from jax.experimental.pallas import tpu_sc as plsc

_N = 100000
_E = 1600000
_DIN = 128
_H = 16
_C = 40

_NC = 2
_NS = 16
_NW = _NC * _NS

_LANE = 128
_KJ = 8
_NB = 50
_RPT = _KJ * _NB
_ROWS_TOTAL = _NW * _RPT
_EP = _ROWS_TOTAL * _LANE
_EPT = _EP // _NW

_NACC = 100096
_SLICE = _NACC // _NS
_RB = _KJ * _LANE


def _zero_fill(ref, nrows):
    zv = jnp.zeros((_H,), jnp.float32)

    def _row(i, carry):
        ref[i, :] = zv
        return carry

    lax.fori_loop(0, nrows, _row, 0)


def _init_acc(acc, rows_v, s):
    z0 = s * _SLICE
    for k in range(_SLICE // _RB):
        pltpu.sync_copy(rows_v, acc.at[pl.ds(z0 + k * _RB, _RB)])
    rem = _SLICE % _RB
    if rem:
        pltpu.sync_copy(rows_v.at[pl.ds(0, rem)],
                        acc.at[pl.ds(z0 + (_SLICE // _RB) * _RB, rem)])


def _seg_body(g_hbm, src_hbm, dst_hbm, out_hbm,
              is0, is1, id0, id1,
              rows_v, acc, semG, semS, semI):
    c = lax.axis_index("c")
    s = lax.axis_index("s")
    w = c * _NS + s
    tile_row0 = w * _RPT
    idxs = (is0, is1)
    idxd = (id0, id1)

    _zero_fill(rows_v, _RB)
    _init_acc(acc, rows_v, s)
    plsc.subcore_barrier()

    def fire_i(b, slot):
        r0 = tile_row0 + b * _KJ
        pltpu.async_copy(src_hbm.at[pl.ds(r0, _KJ)], idxs[slot], semI)
        pltpu.async_copy(dst_hbm.at[pl.ds(r0, _KJ)], idxd[slot], semI)

    def drain_i(slot):
        pltpu.make_async_copy(src_hbm.at[pl.ds(0, _KJ)], idxs[slot], semI).wait()
        pltpu.make_async_copy(dst_hbm.at[pl.ds(0, _KJ)], idxd[slot], semI).wait()

    def fire_g(slot):
        for j in range(_KJ):
            pltpu.async_copy(g_hbm.at[idxs[slot].at[j]],
                             rows_v.at[pl.ds(j * _LANE, _LANE)], semG)

    def drain_g():
        for j in range(_KJ):
            pltpu.make_async_copy(g_hbm.at[pl.ds(0, _LANE)],
                                  rows_v.at[pl.ds(j * _LANE, _LANE)],
                                  semG).wait()

    def fire_s(slot):
        for j in range(_KJ):
            pltpu.async_copy(rows_v.at[pl.ds(j * _LANE, _LANE)],
                             acc.at[idxd[slot].at[j]], semS, add=True)

    def drain_s():
        for j in range(_KJ):
            pltpu.make_async_copy(rows_v.at[pl.ds(j * _LANE, _LANE)],
                                  acc.at[pl.ds(0, _LANE)], semS).wait()

    def _burst(gi, carry):
        r0 = tile_row0 + gi * _KJ
        pltpu.sync_copy(src_hbm.at[pl.ds(r0, _KJ)], idxs[0])
        pltpu.sync_copy(dst_hbm.at[pl.ds(r0, _KJ)], idxd[0])
        cps = [
            pltpu.async_copy(g_hbm.at[idxs[0].at[j]],
                             rows_v.at[pl.ds(j * _LANE, _LANE)], semG)
            for j in range(_KJ)
        ]
        for cp in cps:
            cp.wait()
        for j in range(_KJ):
            pltpu.sync_copy(rows_v.at[pl.ds(j * _LANE, _LANE)],
                            acc.at[idxd[0].at[j]], add=True)
        return carry

    lax.fori_loop(0, _NB, _burst, 0)

    plsc.subcore_barrier()
    pltpu.sync_copy(acc.at[pl.ds(s * _SLICE, _SLICE)],
                    out_hbm.at[c, pl.ds(s * _SLICE, _SLICE)])


_seg_sum = pl.kernel(
    _seg_body,
    out_type=jax.ShapeDtypeStruct((_NC, _NACC, _H), jnp.float32),
    mesh=plsc.VectorSubcoreMesh(core_axis_name="c", subcore_axis_name="s",
                                num_cores=_NC, num_subcores=_NS),
    scratch_types=(
        [pltpu.VMEM((_KJ, _LANE), jnp.int32) for _ in range(4)]
        + [
            pltpu.VMEM((_RB, _H), jnp.float32),
            pltpu.VMEM_SHARED((_NACC, _H), jnp.float32),
            pltpu.SemaphoreType.DMA,
            pltpu.SemaphoreType.DMA,
            pltpu.SemaphoreType.DMA,
        ]
    ),
    compiler_params=pltpu.CompilerParams(use_tc_tiling_on_sc=False),
)


_CH = 10
_NCH = _RPT // _CH
_TDEG = (_NCH - 1) // 3


def _deg_body(dst_hbm, out_hbm, id0, id1, id2, rows_v, acc, semS, semI):
    c = lax.axis_index("c")
    s = lax.axis_index("s")
    w = c * _NS + s
    tile_row0 = w * _RPT
    idxd = (id0, id1, id2)

    _zero_fill(rows_v, _RB)
    _init_acc(acc, rows_v, s)

    ov = jnp.ones((_H,), jnp.float32)

    def _ones_row(i, carry):
        rows_v[i, :] = ov
        return carry

    lax.fori_loop(0, _LANE, _ones_row, 0)
    plsc.subcore_barrier()

    ones_blk = rows_v.at[pl.ds(0, _LANE)]

    def fire_i(ch, slot):
        r0 = jnp.minimum(ch, _NCH - 1) * _CH + tile_row0
        pltpu.async_copy(dst_hbm.at[pl.ds(r0, _CH)], idxd[slot], semI)

    def drain_i(slot):
        pltpu.make_async_copy(dst_hbm.at[pl.ds(0, _CH)], idxd[slot], semI).wait()

    def fire_s(slot):
        for j in range(_CH):
            pltpu.async_copy(ones_blk, acc.at[idxd[slot].at[j]], semS, add=True)

    def drain_s():
        for j in range(_CH):
            pltpu.make_async_copy(ones_blk, acc.at[pl.ds(0, _LANE)], semS).wait()

    fire_i(0, 0)
    fire_i(1, 1)
    drain_i(0)
    fire_s(0)
    fire_i(2, 2)

    def _trip(t, carry):
        for k in range(3):
            ch = 3 * t + 1 + k
            slot = (1 + k) % 3
            drain_i(slot)
            fire_s(slot)
            drain_s()
            fire_i(ch + 2, (slot + 2) % 3)
        return carry

    lax.fori_loop(0, _TDEG, _trip, 0)

    drain_s()
    drain_i(0)
    drain_i(1)

    plsc.subcore_barrier()
    pltpu.sync_copy(acc.at[pl.ds(s * _SLICE, _SLICE)],
                    out_hbm.at[c, pl.ds(s * _SLICE, _SLICE)])


_deg_sum = pl.kernel(
    _deg_body,
    out_type=jax.ShapeDtypeStruct((_NC, _NACC, _H), jnp.float32),
    mesh=plsc.VectorSubcoreMesh(core_axis_name="c", subcore_axis_name="s",
                                num_cores=_NC, num_subcores=_NS),
    scratch_types=[
        pltpu.VMEM((_CH, _LANE), jnp.int32),
        pltpu.VMEM((_CH, _LANE), jnp.int32),
        pltpu.VMEM((_CH, _LANE), jnp.int32),
        pltpu.VMEM((_RB, _H), jnp.float32),
        pltpu.VMEM_SHARED((_NACC, _H), jnp.float32),
        pltpu.SemaphoreType.DMA,
        pltpu.SemaphoreType.DMA,
    ],
    compiler_params=pltpu.CompilerParams(use_tc_tiling_on_sc=False),
)


_BN = 2000
_GRID = _N // _BN


def _tca_body(x_ref, w1_ref, deg_ref, g1_ref, dx_ref):
    cnt = deg_ref[0, :, 0] + deg_ref[1, :, 0]
    dinv = lax.rsqrt(cnt + 1.0)
    h = jnp.dot(x_ref[...], w1_ref[...], preferred_element_type=jnp.float32)
    g1_ref[...] = dinv[:, None] * h
    dx_ref[...] = jnp.broadcast_to(dinv[:, None], (_BN, _H))


def _tcb_body(s1_ref, g1_ref, dx_ref, b1_ref, g2_ref):
    dx = dx_ref[...]
    a1 = dx * (s1_ref[0] + s1_ref[1] + g1_ref[...]) + b1_ref[...]
    g2_ref[...] = dx * jnp.maximum(a1, 0.0)


def _tcc_body(s2_ref, g2_ref, dx_ref, w2_ref, b2_ref, out_ref):
    a2 = dx_ref[...] * (s2_ref[0] + s2_ref[1] + g2_ref[...])
    o = jnp.dot(a2, w2_ref[...], preferred_element_type=jnp.float32) + b2_ref[...]
    m = jnp.max(o, axis=1, keepdims=True)
    lse = jnp.log(jnp.sum(jnp.exp(o - m), axis=1, keepdims=True)) + m
    out_ref[...] = o - lse


def _tca(x, W1, deg2):
    return pl.pallas_call(
        _tca_body,
        grid=(_GRID,),
        in_specs=[
            pl.BlockSpec((_BN, _DIN), lambda i: (i, 0)),
            pl.BlockSpec((_DIN, _H), lambda i: (0, 0)),
            pl.BlockSpec((_NC, _BN, _H), lambda i: (0, i, 0)),
        ],
        out_specs=[
            pl.BlockSpec((_BN, _H), lambda i: (i, 0)),
            pl.BlockSpec((_BN, _H), lambda i: (i, 0)),
        ],
        out_shape=[
            jax.ShapeDtypeStruct((_N, _H), jnp.float32),
            jax.ShapeDtypeStruct((_N, _H), jnp.float32),
        ],
    )(x, W1, deg2)


def _tcb(s1, g1, dx, b1):
    return pl.pallas_call(
        _tcb_body,
        grid=(_GRID,),
        in_specs=[
            pl.BlockSpec((_NC, _BN, _H), lambda i: (0, i, 0)),
            pl.BlockSpec((_BN, _H), lambda i: (i, 0)),
            pl.BlockSpec((_BN, _H), lambda i: (i, 0)),
            pl.BlockSpec((_H,), lambda i: (0,)),
        ],
        out_specs=pl.BlockSpec((_BN, _H), lambda i: (i, 0)),
        out_shape=jax.ShapeDtypeStruct((_N, _H), jnp.float32),
    )(s1, g1, dx, b1)


def _tcc(s2, g2, dx, W2, b2):
    return pl.pallas_call(
        _tcc_body,
        grid=(_GRID,),
        in_specs=[
            pl.BlockSpec((_NC, _BN, _H), lambda i: (0, i, 0)),
            pl.BlockSpec((_BN, _H), lambda i: (i, 0)),
            pl.BlockSpec((_BN, _H), lambda i: (i, 0)),
            pl.BlockSpec((_H, _C), lambda i: (0, 0)),
            pl.BlockSpec((_C,), lambda i: (0,)),
        ],
        out_specs=pl.BlockSpec((_BN, _C), lambda i: (i, 0)),
        out_shape=jax.ShapeDtypeStruct((_N, _C), jnp.float32),
    )(s2, g2, dx, W2, b2)


def kernel(x, edge_index, W1, b1, W2, b2):
    src = edge_index[0]
    dst = edge_index[1]
    pad = _EP - _E
    srcp = jnp.concatenate(
        [src, jnp.zeros((pad,), jnp.int32)]).reshape(_ROWS_TOTAL, _LANE)
    dstp = jnp.concatenate(
        [dst, jnp.full((pad,), _N, jnp.int32)]).reshape(_ROWS_TOTAL, _LANE)

    deg2 = _deg_sum(dstp)
    g1, dx = _tca(x, W1, deg2)
    s1 = _seg_sum(g1, srcp, dstp)
    g2 = _tcb(s1, g1, dx, b1)
    s2 = _seg_sum(g2, srcp, dstp)
    return _tcc(s2, g2, dx, W2, b2)


# speedup vs baseline: 1.4775x; 1.4775x over previous
import jax
import jax.numpy as jnp
from jax import lax
from jax.experimental import pallas as pl
from jax.experimental.pallas import tpu as pltpu
from jax.experimental.pallas import tpu_sc as plsc

_N = 100000
_E = 1600000
_DIN = 128
_H = 16
_C = 40

_NC = 2
_NS = 16
_NW = _NC * _NS

_LANE = 128
_KJ = 5
_NB = 80
_RPT = _KJ * _NB
_ROWS_TOTAL = _NW * _RPT
_EP = _ROWS_TOTAL * _LANE
_EPT = _EP // _NW

_NACC = 100096
_SLICE = _NACC // _NS
_RB = _KJ * _LANE


def _zero_fill(ref, nrows):
    zv = jnp.zeros((_H,), jnp.float32)

    def _row(i, carry):
        ref[i, :] = zv
        return carry

    lax.fori_loop(0, nrows, _row, 0)


def _init_acc(acc, rows_v, s):
    z0 = s * _SLICE
    for k in range(_SLICE // _RB):
        pltpu.sync_copy(rows_v, acc.at[pl.ds(z0 + k * _RB, _RB)])
    rem = _SLICE % _RB
    if rem:
        pltpu.sync_copy(rows_v.at[pl.ds(0, rem)],
                        acc.at[pl.ds(z0 + (_SLICE // _RB) * _RB, rem)])


def _seg_body(g_hbm, src_hbm, dst_hbm, out_hbm,
              is0, is1, is2, is3, id0, id1, id2, id3,
              rows0, rows1, acc, semG, semS, semI):
    c = lax.axis_index("c")
    s = lax.axis_index("s")
    w = c * _NS + s
    tile_row0 = w * _RPT
    rows = (rows0, rows1)
    idxs = (is0, is1, is2, is3)
    idxd = (id0, id1, id2, id3)

    _zero_fill(rows0, _RB)
    _init_acc(acc, rows0, s)
    plsc.subcore_barrier()

    def fire_i(b, slot):
        r0 = tile_row0 + b * _KJ
        pltpu.async_copy(src_hbm.at[pl.ds(r0, _KJ)], idxs[slot], semI)
        pltpu.async_copy(dst_hbm.at[pl.ds(r0, _KJ)], idxd[slot], semI)

    def drain_i(slot):
        pltpu.make_async_copy(src_hbm.at[pl.ds(0, _KJ)], idxs[slot], semI).wait()
        pltpu.make_async_copy(dst_hbm.at[pl.ds(0, _KJ)], idxd[slot], semI).wait()

    def fire_g(slot, buf):
        for j in range(_KJ):
            pltpu.async_copy(g_hbm.at[idxs[slot].at[j]],
                             rows[buf].at[pl.ds(j * _LANE, _LANE)], semG)

    def drain_g(buf):
        for j in range(_KJ):
            pltpu.make_async_copy(g_hbm.at[pl.ds(0, _LANE)],
                                  rows[buf].at[pl.ds(j * _LANE, _LANE)],
                                  semG).wait()

    def fire_s(slot, buf):
        for j in range(_KJ):
            pltpu.async_copy(rows[buf].at[pl.ds(j * _LANE, _LANE)],
                             acc.at[idxd[slot].at[j]], semS, add=True)

    def drain_s(buf):
        for j in range(_KJ):
            pltpu.make_async_copy(rows[buf].at[pl.ds(j * _LANE, _LANE)],
                                  acc.at[pl.ds(0, _LANE)], semS).wait()

    fire_i(0, 0)
    fire_i(1, 1)
    drain_i(0)
    fire_g(0, 0)
    drain_i(1)
    drain_g(0)
    fire_s(0, 0)
    fire_g(1, 1)
    fire_i(2, 2)
    drain_i(2)
    drain_g(1)
    fire_s(1, 1)
    drain_s(0)
    fire_g(2, 0)
    fire_i(3, 3)

    def _quad(t, carry):
        for k in range(4):
            b = 4 * t + 2 + k
            drain_i((3 + k) % 4)
            drain_g(k % 2)
            fire_s((2 + k) % 4, k % 2)
            drain_s((1 + k) % 2)
            fire_g((3 + k) % 4, (1 + k) % 2)
            fire_i(b + 2, k % 4)
        return carry

    lax.fori_loop(0, (_NB - 4) // 4, _quad, 0)

    drain_i(3)
    drain_g(0)
    fire_s(2, 0)
    drain_s(1)
    fire_g(3, 1)
    drain_g(1)
    fire_s(3, 1)
    drain_s(0)
    drain_s(1)

    plsc.subcore_barrier()
    pltpu.sync_copy(acc.at[pl.ds(s * _SLICE, _SLICE)],
                    out_hbm.at[c, pl.ds(s * _SLICE, _SLICE)])


_seg_sum = pl.kernel(
    _seg_body,
    out_type=jax.ShapeDtypeStruct((_NC, _NACC, _H), jnp.float32),
    mesh=plsc.VectorSubcoreMesh(core_axis_name="c", subcore_axis_name="s",
                                num_cores=_NC, num_subcores=_NS),
    scratch_types=(
        [pltpu.VMEM((_KJ, _LANE), jnp.int32) for _ in range(8)]
        + [
            pltpu.VMEM((_RB, _H), jnp.float32),
            pltpu.VMEM((_RB, _H), jnp.float32),
            pltpu.VMEM_SHARED((_NACC, _H), jnp.float32),
            pltpu.SemaphoreType.DMA,
            pltpu.SemaphoreType.DMA,
            pltpu.SemaphoreType.DMA,
        ]
    ),
    compiler_params=pltpu.CompilerParams(use_tc_tiling_on_sc=False),
)


_CH = 10
_NCH = _RPT // _CH
_TDEG = (_NCH - 1) // 3


def _deg_body(dst_hbm, out_hbm, id0, id1, id2, rows_v, acc, semS, semI):
    c = lax.axis_index("c")
    s = lax.axis_index("s")
    w = c * _NS + s
    tile_row0 = w * _RPT
    idxd = (id0, id1, id2)

    _zero_fill(rows_v, _RB)
    _init_acc(acc, rows_v, s)

    ov = jnp.ones((_H,), jnp.float32)

    def _ones_row(i, carry):
        rows_v[i, :] = ov
        return carry

    lax.fori_loop(0, _LANE, _ones_row, 0)
    plsc.subcore_barrier()

    ones_blk = rows_v.at[pl.ds(0, _LANE)]

    def fire_i(ch, slot):
        r0 = jnp.minimum(ch, _NCH - 1) * _CH + tile_row0
        pltpu.async_copy(dst_hbm.at[pl.ds(r0, _CH)], idxd[slot], semI)

    def drain_i(slot):
        pltpu.make_async_copy(dst_hbm.at[pl.ds(0, _CH)], idxd[slot], semI).wait()

    def fire_s(slot):
        for j in range(_CH):
            pltpu.async_copy(ones_blk, acc.at[idxd[slot].at[j]], semS, add=True)

    def drain_s():
        for j in range(_CH):
            pltpu.make_async_copy(ones_blk, acc.at[pl.ds(0, _LANE)], semS).wait()

    fire_i(0, 0)
    fire_i(1, 1)
    drain_i(0)
    fire_s(0)
    fire_i(2, 2)

    def _trip(t, carry):
        for k in range(3):
            ch = 3 * t + 1 + k
            slot = (1 + k) % 3
            drain_i(slot)
            fire_s(slot)
            drain_s()
            fire_i(ch + 2, (slot + 2) % 3)
        return carry

    lax.fori_loop(0, _TDEG, _trip, 0)

    drain_s()
    drain_i(0)
    drain_i(1)

    plsc.subcore_barrier()
    pltpu.sync_copy(acc.at[pl.ds(s * _SLICE, _SLICE)],
                    out_hbm.at[c, pl.ds(s * _SLICE, _SLICE)])


_deg_sum = pl.kernel(
    _deg_body,
    out_type=jax.ShapeDtypeStruct((_NC, _NACC, _H), jnp.float32),
    mesh=plsc.VectorSubcoreMesh(core_axis_name="c", subcore_axis_name="s",
                                num_cores=_NC, num_subcores=_NS),
    scratch_types=[
        pltpu.VMEM((_CH, _LANE), jnp.int32),
        pltpu.VMEM((_CH, _LANE), jnp.int32),
        pltpu.VMEM((_CH, _LANE), jnp.int32),
        pltpu.VMEM((_RB, _H), jnp.float32),
        pltpu.VMEM_SHARED((_NACC, _H), jnp.float32),
        pltpu.SemaphoreType.DMA,
        pltpu.SemaphoreType.DMA,
    ],
    compiler_params=pltpu.CompilerParams(use_tc_tiling_on_sc=False),
)


_PK = 16
_NPR = _N // _PK
_NPRA = _NACC // _PK
_PH = _PK * _H
_PX = _PK * _DIN
_PC = _PK * _C
_BNP = 512
_GRID = (_NPR + _BNP - 1) // _BNP


def _tca_body(xv_ref, w1bd_ref, degv_ref, g1p_ref, dxp_ref):
    dinvp = lax.rsqrt(degv_ref[0] + degv_ref[1] + 1.0)
    hp = jnp.dot(xv_ref[...], w1bd_ref[...],
                 preferred_element_type=jnp.float32)
    g1p_ref[...] = dinvp * hp
    dxp_ref[...] = dinvp


def _tcb_body(s1v_ref, g1p_ref, dxp_ref, b1t_ref, g2p_ref):
    dxp = dxp_ref[...]
    a1 = dxp * (s1v_ref[0] + s1v_ref[1] + g1p_ref[...]) + b1t_ref[...]
    g2p_ref[...] = dxp * jnp.maximum(a1, 0.0)


def _tcc_body(s2v_ref, g2p_ref, dxp_ref, w2bd_ref, b2t_ref, out_ref):
    a2p = dxp_ref[...] * (s2v_ref[0] + s2v_ref[1] + g2p_ref[...])
    o = jnp.dot(a2p, w2bd_ref[...],
                preferred_element_type=jnp.float32) + b2t_ref[...]
    parts = []
    for a in range(_PK):
        sl = o[:, a * _C:(a + 1) * _C]
        m = jnp.max(sl, axis=1, keepdims=True)
        lse = jnp.log(jnp.sum(jnp.exp(sl - m), axis=1, keepdims=True)) + m
        parts.append(sl - lse)
    out_ref[...] = jnp.concatenate(parts, axis=1)


def _tca(xv, W1bd, degv):
    return pl.pallas_call(
        _tca_body,
        grid=(_GRID,),
        in_specs=[
            pl.BlockSpec((_BNP, _PX), lambda i: (i, 0)),
            pl.BlockSpec((_PX, _PH), lambda i: (0, 0)),
            pl.BlockSpec((_NC, _BNP, _PH), lambda i: (0, i, 0)),
        ],
        out_specs=[
            pl.BlockSpec((_BNP, _PH), lambda i: (i, 0)),
            pl.BlockSpec((_BNP, _PH), lambda i: (i, 0)),
        ],
        out_shape=[
            jax.ShapeDtypeStruct((_NPR, _PH), jnp.float32),
            jax.ShapeDtypeStruct((_NPR, _PH), jnp.float32),
        ],
    )(xv, W1bd, degv)


def _tcb(s1v, g1p, dxp, b1t):
    return pl.pallas_call(
        _tcb_body,
        grid=(_GRID,),
        in_specs=[
            pl.BlockSpec((_NC, _BNP, _PH), lambda i: (0, i, 0)),
            pl.BlockSpec((_BNP, _PH), lambda i: (i, 0)),
            pl.BlockSpec((_BNP, _PH), lambda i: (i, 0)),
            pl.BlockSpec((_PH,), lambda i: (0,)),
        ],
        out_specs=pl.BlockSpec((_BNP, _PH), lambda i: (i, 0)),
        out_shape=jax.ShapeDtypeStruct((_NPR, _PH), jnp.float32),
    )(s1v, g1p, dxp, b1t)


def _tcc(s2v, g2p, dxp, W2bd, b2t):
    return pl.pallas_call(
        _tcc_body,
        grid=(_GRID,),
        in_specs=[
            pl.BlockSpec((_NC, _BNP, _PH), lambda i: (0, i, 0)),
            pl.BlockSpec((_BNP, _PH), lambda i: (i, 0)),
            pl.BlockSpec((_BNP, _PH), lambda i: (i, 0)),
            pl.BlockSpec((_PH, _PC), lambda i: (0, 0)),
            pl.BlockSpec((_PC,), lambda i: (0,)),
        ],
        out_specs=pl.BlockSpec((_BNP, _PC), lambda i: (i, 0)),
        out_shape=jax.ShapeDtypeStruct((_NPR, _PC), jnp.float32),
    )(s2v, g2p, dxp, W2bd, b2t)


def _block_diag(W, copies):
    k, m = W.shape
    eye = jnp.eye(copies, dtype=W.dtype)
    return (eye[:, None, :, None] * W[None, :, None, :]).reshape(
        copies * k, copies * m)


def kernel(x, edge_index, W1, b1, W2, b2):
    src = edge_index[0]
    dst = edge_index[1]
    pad = _EP - _E
    srcp = jnp.concatenate(
        [src, jnp.zeros((pad,), jnp.int32)]).reshape(_ROWS_TOTAL, _LANE)
    dstp = jnp.concatenate(
        [dst, jnp.full((pad,), _N, jnp.int32)]).reshape(_ROWS_TOTAL, _LANE)

    xv = x.reshape(_NPR, _PX)
    W1bd = _block_diag(W1, _PK)
    W2bd = _block_diag(W2, _PK)
    b1t = jnp.tile(b1, _PK)
    b2t = jnp.tile(b2, _PK)

    deg2 = _deg_sum(dstp)
    degv = deg2.reshape(_NC, _NPRA, _PH)
    g1p, dxp = _tca(xv, W1bd, degv)
    s1 = _seg_sum(g1p.reshape(_N, _H), srcp, dstp)
    g2p = _tcb(s1.reshape(_NC, _NPRA, _PH), g1p, dxp, b1t)
    s2 = _seg_sum(g2p.reshape(_N, _H), srcp, dstp)
    outp = _tcc(s2.reshape(_NC, _NPRA, _PH), g2p, dxp, W2bd, b2t)
    return outp.reshape(_N, _C)
